# baseline (speedup 1.0000x reference)
import functools

import jax
import jax.numpy as jnp
from jax import lax
from jax.experimental import pallas as pl
from jax.experimental.pallas import tpu as pltpu
from jax.experimental.pallas import tpu_sc as plsc

N_EMB = 8192
EMB_DIM = 32
BETA = 0.25
B, T = 16, 1024
M_TOTAL = B * T
M_BLK = 512
N_BLKS = M_TOTAL // M_BLK
K_CHUNK = 4096
N_CHUNKS = N_EMB // K_CHUNK

NW = 32
ROWS_PER_W = M_TOTAL // NW
IDX_CHUNK = 128
N_IDX_CHUNKS = ROWS_PER_W // IDX_CHUNK


def _vq_tc_body(z_ref, emb_ref, bsq_ref, idx_ref, loss_ref):
    z = z_ref[...]
    a = jnp.sum(z * z, axis=1, keepdims=True)
    carry = jnp.full((M_BLK, 1), jnp.inf, dtype=jnp.float32)
    run_val = jnp.zeros((M_BLK, 1), dtype=jnp.float32)
    run_idx = jnp.zeros((M_BLK, 1), dtype=jnp.int32)
    for c in range(N_CHUNKS):
        e2 = emb_ref[pl.ds(c * K_CHUNK, K_CHUNK), :]
        m2 = lax.dot_general(z, e2, (((1,), (1,)), ((), ())),
                             preferred_element_type=jnp.float32)
        b = bsq_ref[:, pl.ds(c * K_CHUNK, K_CHUNK)]
        d = (a + b) - m2
        lmin = jnp.min(d, axis=1, keepdims=True)
        ii = lax.broadcasted_iota(jnp.int32, d.shape, 1)
        lidx = jnp.min(jnp.where(d == lmin, ii, K_CHUNK),
                       axis=1, keepdims=True) + c * K_CHUNK
        take = lmin < carry
        carry = jnp.where(take, lmin, carry).astype(jnp.bfloat16).astype(
            jnp.float32)
        run_val = jnp.where(take, lmin, run_val)
        run_idx = jnp.where(take, lidx, run_idx)
    idx_ref[...] = run_idx

    @pl.when(pl.program_id(0) == 0)
    def _():
        loss_ref[...] = jnp.zeros((1, 1), jnp.float32)

    loss_ref[...] = loss_ref[...] + jnp.sum(run_val)[None, None]


_vq_tc = pl.pallas_call(
    _vq_tc_body,
    grid=(N_BLKS,),
    in_specs=[
        pl.BlockSpec((M_BLK, EMB_DIM), lambda i: (i, 0)),
        pl.BlockSpec((N_EMB, EMB_DIM), lambda i: (0, 0)),
        pl.BlockSpec((1, N_EMB), lambda i: (0, 0)),
    ],
    out_specs=[
        pl.BlockSpec((M_BLK, 1), lambda i: (i, 0)),
        pl.BlockSpec((1, 1), lambda i: (0, 0)),
    ],
    out_shape=[
        jax.ShapeDtypeStruct((M_TOTAL, 1), jnp.int32),
        jax.ShapeDtypeStruct((1, 1), jnp.float32),
    ],
    compiler_params=pltpu.CompilerParams(
        dimension_semantics=("arbitrary",),
    ),
)


@functools.cache
def _make_sc_gather():
    mesh = plsc.VectorSubcoreMesh(core_axis_name="c", subcore_axis_name="s")

    @functools.partial(
        pl.kernel,
        mesh=mesh,
        out_type=jax.ShapeDtypeStruct((M_TOTAL, EMB_DIM), jnp.float32),
        scratch_types=[
            pltpu.VMEM((N_IDX_CHUNKS, IDX_CHUNK), jnp.int32),
            pltpu.VMEM((ROWS_PER_W, EMB_DIM), jnp.float32),
            pltpu.SemaphoreType.DMA,
        ],
        compiler_params=pltpu.CompilerParams(use_tc_tiling_on_sc=False),
    )
    def sc_gather(table_hbm, idx_hbm, out_hbm, idx_v, rows_v, sem):
        wid = lax.axis_index("s") * 2 + lax.axis_index("c")
        base = wid * ROWS_PER_W
        pltpu.sync_copy(idx_hbm.at[wid], idx_v)
        for k in range(N_IDX_CHUNKS):
            pltpu.async_copy(
                table_hbm.at[idx_v.at[k]],
                rows_v.at[pl.ds(k * IDX_CHUNK, IDX_CHUNK)],
                sem,
            ).wait()
        pltpu.sync_copy(rows_v, out_hbm.at[pl.ds(base, ROWS_PER_W)])

    return sc_gather


def kernel(z, embedding_weight):
    zp = jnp.transpose(z, (0, 2, 1))
    z_flat = zp.reshape(M_TOTAL, EMB_DIM)
    bsq = jnp.sum(embedding_weight ** 2, axis=1)[None, :]

    idx2, loss_sum = _vq_tc(z_flat, embedding_weight * 2.0, bsq)
    indices = idx2.reshape(M_TOTAL)

    idx_sc = indices.reshape(NW, N_IDX_CHUNKS, IDX_CHUNK)
    zq_flat = _make_sc_gather()(embedding_weight, idx_sc)

    zq = zq_flat.reshape(B, T, EMB_DIM)
    zq_st = zp + (zq - zp)
    out = jnp.transpose(zq_st, (0, 2, 1))
    loss = (loss_sum[0, 0] / (M_TOTAL * EMB_DIM)) * (1.0 + BETA)
    return (out, loss, (None, None, indices))


# speedup vs baseline: 1.0819x; 1.0819x over previous
import functools

import jax
import jax.numpy as jnp
from jax import lax
from jax.experimental import pallas as pl
from jax.experimental.pallas import tpu as pltpu
from jax.experimental.pallas import tpu_sc as plsc

N_EMB = 8192
EMB_DIM = 32
BETA = 0.25
B, T = 16, 1024
M_TOTAL = B * T
M_BLK = 512
N_BLKS = M_TOTAL // M_BLK
K_CHUNK = 4096
N_CHUNKS = N_EMB // K_CHUNK

NW = 32
ROWS_PER_W = M_TOTAL // NW
IDX_CHUNK = 128
N_IDX_CHUNKS = ROWS_PER_W // IDX_CHUNK


def _vq_tc_body(z_ref, emb_ref, bsq_ref, idx_ref, loss_ref):
    z = z_ref[...]
    a = jnp.sum(z * z, axis=1, keepdims=True)
    carry = jnp.full((M_BLK, 1), jnp.inf, dtype=jnp.float32)
    run_val = jnp.zeros((M_BLK, 1), dtype=jnp.float32)
    run_idx = jnp.zeros((M_BLK, 1), dtype=jnp.int32)
    for c in range(N_CHUNKS):
        e = emb_ref[pl.ds(c * K_CHUNK, K_CHUNK), :]
        m = lax.dot_general(z, e, (((1,), (1,)), ((), ())),
                            preferred_element_type=jnp.float32)
        b = bsq_ref[:, pl.ds(c * K_CHUNK, K_CHUNK)]
        d = (a + b) - 2.0 * m
        lmin = jnp.min(d, axis=1, keepdims=True)
        ii = lax.broadcasted_iota(jnp.int32, d.shape, 1)
        lidx = jnp.min(jnp.where(d == lmin, ii, K_CHUNK),
                       axis=1, keepdims=True) + c * K_CHUNK
        take = lmin < carry
        carry = jnp.where(take, lmin, carry).astype(jnp.bfloat16).astype(
            jnp.float32)
        run_val = jnp.where(take, lmin, run_val)
        run_idx = jnp.where(take, lidx, run_idx)
    idx_ref[...] = run_idx

    @pl.when(pl.program_id(0) == 0)
    def _():
        loss_ref[...] = jnp.zeros((1, 1), jnp.float32)

    loss_ref[...] = loss_ref[...] + jnp.sum(run_val)[None, None]


_vq_tc = pl.pallas_call(
    _vq_tc_body,
    grid=(N_BLKS,),
    in_specs=[
        pl.BlockSpec((M_BLK, EMB_DIM), lambda i: (i, 0)),
        pl.BlockSpec((N_EMB, EMB_DIM), lambda i: (0, 0)),
        pl.BlockSpec((1, N_EMB), lambda i: (0, 0)),
    ],
    out_specs=[
        pl.BlockSpec((M_BLK, 1), lambda i: (i, 0)),
        pl.BlockSpec((1, 1), lambda i: (0, 0)),
    ],
    out_shape=[
        jax.ShapeDtypeStruct((M_TOTAL, 1), jnp.int32),
        jax.ShapeDtypeStruct((1, 1), jnp.float32),
    ],
    compiler_params=pltpu.CompilerParams(
        dimension_semantics=("arbitrary",),
    ),
)


@functools.cache
def _make_sc_gather():
    mesh = plsc.VectorSubcoreMesh(core_axis_name="c", subcore_axis_name="s")

    @functools.partial(
        pl.kernel,
        mesh=mesh,
        out_type=jax.ShapeDtypeStruct((M_TOTAL, EMB_DIM), jnp.float32),
        scratch_types=[
            pltpu.VMEM((N_IDX_CHUNKS, IDX_CHUNK), jnp.int32),
            pltpu.VMEM((ROWS_PER_W, EMB_DIM), jnp.float32),
            pltpu.SemaphoreType.DMA,
        ],
        compiler_params=pltpu.CompilerParams(use_tc_tiling_on_sc=False),
    )
    def sc_gather(table_hbm, idx_hbm, out_hbm, idx_v, rows_v, sem):
        wid = lax.axis_index("s") * 2 + lax.axis_index("c")
        base = wid * ROWS_PER_W
        pltpu.sync_copy(idx_hbm.at[wid], idx_v)
        for k in range(N_IDX_CHUNKS):
            pltpu.async_copy(
                table_hbm.at[idx_v.at[k]],
                rows_v.at[pl.ds(k * IDX_CHUNK, IDX_CHUNK)],
                sem,
            ).wait()
        pltpu.sync_copy(rows_v, out_hbm.at[pl.ds(base, ROWS_PER_W)])

    return sc_gather


def kernel(z, embedding_weight):
    zp = jnp.transpose(z, (0, 2, 1))
    z_flat = zp.reshape(M_TOTAL, EMB_DIM)
    bsq = jnp.sum(embedding_weight ** 2, axis=1)[None, :]

    idx2, loss_sum = _vq_tc(z_flat, embedding_weight, bsq)
    indices = idx2.reshape(M_TOTAL)

    idx_sc = indices.reshape(NW, N_IDX_CHUNKS, IDX_CHUNK)
    zq_flat = _make_sc_gather()(embedding_weight, idx_sc)

    zq = zq_flat.reshape(B, T, EMB_DIM)
    zq_st = zp + (zq - zp)
    out = jnp.transpose(zq_st, (0, 2, 1))
    loss = (loss_sum[0, 0] / (M_TOTAL * EMB_DIM)) * (1.0 + BETA)
    return (out, loss, (None, None, indices))
